# Initial kernel scaffold; baseline (speedup 1.0000x reference)
import functools

import jax
import jax.numpy as jnp
from jax import lax
from jax.experimental import pallas as pl
from jax.experimental.pallas import tpu as pltpu
from jax.experimental.pallas import tpu_sc as plsc

N_NODES = 100000
D = 32
H = 16
E = 1600000
CHUNK = 128
NBUF = 4
P_CHUNKS_PER_TEC = 784
E_PAD = P_CHUNKS_PER_TEC * 16 * CHUNK
DEG_CHUNKS_PER_TEC = E_PAD // (32 * CHUNK)
ACC_ROWS = N_NODES + 16
ZROWS = 893
ROWS_PER_TEC = ACC_ROWS // 16
OUT_ROWS_PER_TEC = N_NODES // 16

_f32 = jnp.float32
_i32 = jnp.int32

_MESH = plsc.VectorSubcoreMesh(core_axis_name="c", subcore_axis_name="s")


def _zero_acc(acc, zbuf, w):
    def zrow(i, carry):
        zbuf[i, :] = jnp.zeros((16,), _f32)
        return carry

    lax.fori_loop(0, ZROWS, zrow, 0)
    base = w * ROWS_PER_TEC
    for k in range(7):
        pltpu.sync_copy(zbuf, acc.at[pl.ds(base + k * ZROWS, ZROWS)])


def _deg_body(ei_ref, out_ref, acc, zbuf, ones_b, rb0, rb1, sem0, sem1):
    c = lax.axis_index("c")
    w = lax.axis_index("s")
    rbs = (rb0, rb1)
    sems = (sem0, sem1)

    _zero_acc(acc, zbuf, w)

    def orow(i, carry):
        ones_b[i, :] = jnp.ones((16,), _f32)
        return carry

    lax.fori_loop(0, CHUNK, orow, 0)
    plsc.subcore_barrier()

    chunk0 = (c * 16 + w) * DEG_CHUNKS_PER_TEC

    def load(t, b):
        e0 = pl.multiple_of((chunk0 + t) * CHUNK, CHUNK)
        pltpu.sync_copy(ei_ref.at[0, pl.ds(e0, CHUNK)], rbs[b])
        pltpu.async_copy(ones_b, acc.at[rbs[b]], sems[b], add=True)

    def wait(b):
        pltpu.make_async_copy(ones_b, acc.at[rbs[b]], sems[b]).wait()

    load(0, 0)
    load(1, 1)

    def outer(t2, carry):
        for b in range(2):
            t = t2 * 2 + b
            wait(b)
            load(t, b)
        return carry

    lax.fori_loop(1, DEG_CHUNKS_PER_TEC // 2, outer, 0)
    wait(0)
    wait(1)
    plsc.subcore_barrier()

    ob = w * OUT_ROWS_PER_TEC
    pltpu.sync_copy(acc.at[pl.ds(ob, OUT_ROWS_PER_TEC)],
                    out_ref.at[c, pl.ds(ob, OUT_ROWS_PER_TEC)])


_deg_call = functools.partial(
    pl.kernel,
    _deg_body,
    out_type=jax.ShapeDtypeStruct((2, N_NODES, H), _f32),
    mesh=_MESH,
    scratch_types=[
        pltpu.VMEM_SHARED((ACC_ROWS, H), _f32),
        pltpu.VMEM((ZROWS, H), _f32),
        pltpu.VMEM((CHUNK, H), _f32),
        pltpu.VMEM((CHUNK,), _i32),
        pltpu.VMEM((CHUNK,), _i32),
        pltpu.SemaphoreType.DMA,
        pltpu.SemaphoreType.DMA,
    ],
)()


def _prop_body(s_ref, ei_ref, out_ref, acc, zbuf,
               cb0, cb1, cb2, cb3, rb0, rb1, rb2, rb3,
               gb0, gb1, gb2, gb3,
               gs0, gs1, gs2, gs3, ss0, ss1, ss2, ss3):
    c = lax.axis_index("c")
    w = lax.axis_index("s")
    cbs = (cb0, cb1, cb2, cb3)
    rbs = (rb0, rb1, rb2, rb3)
    gbs = (gb0, gb1, gb2, gb3)
    gsems = (gs0, gs1, gs2, gs3)
    ssems = (ss0, ss1, ss2, ss3)

    _zero_acc(acc, zbuf, w)
    plsc.subcore_barrier()

    coff = c * N_NODES
    chunk0 = w * P_CHUNKS_PER_TEC

    def load_gather(j, b):
        e0 = pl.multiple_of((chunk0 + j) * CHUNK, CHUNK)
        pltpu.sync_copy(ei_ref.at[1, pl.ds(e0, CHUNK)], cbs[b])
        for k in range(CHUNK // 16):
            sl = pl.ds(k * 16, 16)
            cbs[b][sl] = cbs[b][sl] + coff
        pltpu.sync_copy(ei_ref.at[0, pl.ds(e0, CHUNK)], rbs[b])
        pltpu.async_copy(s_ref.at[cbs[b]], gbs[b], gsems[b])

    def wait_gather(b):
        pltpu.make_async_copy(s_ref.at[cbs[b]], gbs[b], gsems[b]).wait()

    def start_scatter(b):
        pltpu.async_copy(gbs[b], acc.at[rbs[b]], ssems[b], add=True)

    def wait_scatter(b):
        pltpu.make_async_copy(gbs[b], acc.at[rbs[b]], ssems[b]).wait()

    load_gather(0, 0)
    for b in range(1, NBUF):
        load_gather(b, b)
        wait_gather(b - 1)
        start_scatter(b - 1)

    def outer(t, carry):
        for b in range(NBUF):
            j = t * NBUF + b
            wait_scatter(b)
            load_gather(j, b)
            pb = (b - 1) % NBUF
            wait_gather(pb)
            start_scatter(pb)
        return carry

    lax.fori_loop(1, P_CHUNKS_PER_TEC // NBUF, outer, 0)
    wait_gather(NBUF - 1)
    start_scatter(NBUF - 1)
    for b in range(NBUF):
        wait_scatter(b)
    plsc.subcore_barrier()

    ob = w * OUT_ROWS_PER_TEC
    pltpu.sync_copy(acc.at[pl.ds(ob, OUT_ROWS_PER_TEC)],
                    out_ref.at[pl.ds(coff + ob, OUT_ROWS_PER_TEC)])


_prop_call = functools.partial(
    pl.kernel,
    _prop_body,
    out_type=jax.ShapeDtypeStruct((2 * N_NODES, H), _f32),
    mesh=_MESH,
    scratch_types=(
        [pltpu.VMEM_SHARED((ACC_ROWS, H), _f32), pltpu.VMEM((ZROWS, H), _f32)]
        + [pltpu.VMEM((CHUNK,), _i32) for _ in range(8)]
        + [pltpu.VMEM((CHUNK, H), _f32) for _ in range(4)]
        + [pltpu.SemaphoreType.DMA for _ in range(8)]
    ),
)()


_BLK = 2000
_NB = N_NODES // _BLK


def _dis_body(p_ref, dis_ref):
    deg = p_ref[0, :, 0:1] + p_ref[1, :, 0:1]
    dis_ref[...] = jnp.where(deg > 0.0, lax.rsqrt(jnp.maximum(deg, 1e-30)), 0.0)


def _tc_dis(p):
    return pl.pallas_call(
        _dis_body,
        grid=(_NB,),
        in_specs=[pl.BlockSpec((2, _BLK, H), lambda i: (0, i, 0))],
        out_specs=pl.BlockSpec((_BLK, 1), lambda i: (i, 0)),
        out_shape=jax.ShapeDtypeStruct((N_NODES, 1), _f32),
    )(p)


def _scale0_body(e_ref, dis_ref, s_ref):
    i = pl.program_id(0)
    half = jnp.where(i < _NB, e_ref[:, 0:H], e_ref[:, H:D])
    s_ref[...] = half * dis_ref[...]


def _tc_scale0(e0, dis):
    return pl.pallas_call(
        _scale0_body,
        grid=(2 * _NB,),
        in_specs=[
            pl.BlockSpec((_BLK, D), lambda i: (i % _NB, 0)),
            pl.BlockSpec((_BLK, 1), lambda i: (i % _NB, 0)),
        ],
        out_specs=pl.BlockSpec((_BLK, H), lambda i: (i, 0)),
        out_shape=jax.ShapeDtypeStruct((2 * N_NODES, H), _f32),
    )(e0, dis)


def _rescale_body(a_ref, dis_ref, s_ref):
    d = dis_ref[...]
    s_ref[...] = a_ref[...] * (d * d)


def _tc_rescale(a, dis):
    return pl.pallas_call(
        _rescale_body,
        grid=(2 * _NB,),
        in_specs=[
            pl.BlockSpec((_BLK, H), lambda i: (i, 0)),
            pl.BlockSpec((_BLK, 1), lambda i: (i % _NB, 0)),
        ],
        out_specs=pl.BlockSpec((_BLK, H), lambda i: (i, 0)),
        out_shape=jax.ShapeDtypeStruct((2 * N_NODES, H), _f32),
    )(a, dis)


def _final_body(w_ref, dis_ref, a1l, a1h, a2l, a2h, a3l, a3h, out_ref):
    d = dis_ref[...]
    lo = (a1l[...] + a2l[...] + a3l[...]) * d
    hi = (a1h[...] + a2h[...] + a3h[...]) * d
    out_ref[...] = (w_ref[...] + jnp.concatenate([lo, hi], axis=1)) * 0.25


def _tc_final(weights, dis, a1, a2, a3, node_off_blocks):
    nb = weights.shape[0] // _BLK
    noff = node_off_blocks

    def lo_map(i):
        return (noff + i, 0)

    def hi_map(i):
        return (_NB + noff + i, 0)

    a_specs = []
    for _ in range(3):
        a_specs.append(pl.BlockSpec((_BLK, H), lo_map))
        a_specs.append(pl.BlockSpec((_BLK, H), hi_map))
    return pl.pallas_call(
        _final_body,
        grid=(nb,),
        in_specs=[
            pl.BlockSpec((_BLK, D), lambda i: (i, 0)),
            pl.BlockSpec((_BLK, 1), lo_map),
        ] + a_specs,
        out_specs=pl.BlockSpec((_BLK, D), lambda i: (i, 0)),
        out_shape=jax.ShapeDtypeStruct((weights.shape[0], D), _f32),
    )(weights, dis, a1, a1, a2, a2, a3, a3)


def kernel(user_weight, item_weight, edge_index):
    e0 = jnp.concatenate([user_weight, item_weight], axis=0)
    npad = E_PAD - E
    pad = jnp.stack([
        jnp.full((npad,), N_NODES, _i32),
        jnp.zeros((npad,), _i32),
    ])
    ei = jnp.concatenate([edge_index.astype(_i32), pad], axis=1)

    p = _deg_call(ei)
    dis = _tc_dis(p)
    s = _tc_scale0(e0, dis)
    a1 = _prop_call(s, ei)
    s = _tc_rescale(a1, dis)
    a2 = _prop_call(s, ei)
    s = _tc_rescale(a2, dis)
    a3 = _prop_call(s, ei)

    user_out = _tc_final(user_weight, dis, a1, a2, a3, 0)
    item_out = _tc_final(item_weight, dis, a1, a2, a3, _NB // 2)
    return (user_out, item_out)


# speedup vs baseline: 12.6037x; 12.6037x over previous
import functools

import jax
import jax.numpy as jnp
from jax import lax
from jax.experimental import pallas as pl
from jax.experimental.pallas import tpu as pltpu
from jax.experimental.pallas import tpu_sc as plsc

N_NODES = 100000
D = 32
H = 16
E = 1600000
CHUNK = 128
NBUF = 4
P_CHUNKS_PER_TEC = 784
E_PAD = P_CHUNKS_PER_TEC * 16 * CHUNK
DEG_CHUNKS_PER_TEC = E_PAD // (32 * CHUNK)
ACC_ROWS = N_NODES + 16
ZROWS = 893
ROWS_PER_TEC = ACC_ROWS // 16
OUT_ROWS_PER_TEC = N_NODES // 16

_f32 = jnp.float32
_i32 = jnp.int32

_MESH = plsc.VectorSubcoreMesh(core_axis_name="c", subcore_axis_name="s")
_SC_PARAMS = pltpu.CompilerParams(use_tc_tiling_on_sc=False)


def _zero_acc(acc, zbuf, w):
    def zrow(i, carry):
        zbuf[i, :] = jnp.zeros((16,), _f32)
        return carry

    lax.fori_loop(0, ZROWS, zrow, 0)
    base = w * ROWS_PER_TEC
    for k in range(7):
        pltpu.sync_copy(zbuf, acc.at[pl.ds(base + k * ZROWS, ZROWS)])


def _deg_body(ei_ref, out_ref, acc, zbuf, ones_b, rb0, rb1, sem0, sem1):
    c = lax.axis_index("c")
    w = lax.axis_index("s")
    rbs = (rb0, rb1)
    sems = (sem0, sem1)

    _zero_acc(acc, zbuf, w)

    def orow(i, carry):
        ones_b[i, :] = jnp.ones((16,), _f32)
        return carry

    lax.fori_loop(0, CHUNK, orow, 0)
    plsc.subcore_barrier()

    chunk0 = (c * 16 + w) * DEG_CHUNKS_PER_TEC

    def load(t, b):
        e0 = pl.multiple_of((chunk0 + t) * CHUNK, CHUNK)
        pltpu.sync_copy(ei_ref.at[0, pl.ds(e0, CHUNK)], rbs[b])
        pltpu.async_copy(ones_b, acc.at[rbs[b]], sems[b], add=True)

    def wait(b):
        pltpu.make_async_copy(ones_b, acc.at[rbs[b]], sems[b]).wait()

    load(0, 0)
    load(1, 1)

    def outer(t2, carry):
        for b in range(2):
            t = t2 * 2 + b
            wait(b)
            load(t, b)
        return carry

    lax.fori_loop(1, DEG_CHUNKS_PER_TEC // 2, outer, 0)
    wait(0)
    wait(1)
    plsc.subcore_barrier()

    ob = w * OUT_ROWS_PER_TEC
    pltpu.sync_copy(acc.at[pl.ds(ob, OUT_ROWS_PER_TEC)],
                    out_ref.at[c, pl.ds(ob, OUT_ROWS_PER_TEC)])


_deg_call = functools.partial(
    pl.kernel,
    _deg_body,
    out_type=jax.ShapeDtypeStruct((2, N_NODES, H), _f32),
    mesh=_MESH,
    scratch_types=[
        pltpu.VMEM_SHARED((ACC_ROWS, H), _f32),
        pltpu.VMEM((ZROWS, H), _f32),
        pltpu.VMEM((CHUNK, H), _f32),
        pltpu.VMEM((CHUNK,), _i32),
        pltpu.VMEM((CHUNK,), _i32),
        pltpu.SemaphoreType.DMA,
        pltpu.SemaphoreType.DMA,
    ],
    compiler_params=_SC_PARAMS,
)()


def _prop_body(s_ref, ei_ref, out_ref, acc, zbuf,
               cb0, cb1, cb2, cb3, rb0, rb1, rb2, rb3,
               gb0, gb1, gb2, gb3,
               gs0, gs1, gs2, gs3, ss0, ss1, ss2, ss3):
    c = lax.axis_index("c")
    w = lax.axis_index("s")
    cbs = (cb0, cb1, cb2, cb3)
    rbs = (rb0, rb1, rb2, rb3)
    gbs = (gb0, gb1, gb2, gb3)
    gsems = (gs0, gs1, gs2, gs3)
    ssems = (ss0, ss1, ss2, ss3)

    _zero_acc(acc, zbuf, w)
    plsc.subcore_barrier()

    coff = c * N_NODES
    chunk0 = w * P_CHUNKS_PER_TEC

    def load_gather(j, b):
        e0 = pl.multiple_of((chunk0 + j) * CHUNK, CHUNK)
        pltpu.sync_copy(ei_ref.at[1, pl.ds(e0, CHUNK)], cbs[b])
        for k in range(CHUNK // 16):
            sl = pl.ds(k * 16, 16)
            cbs[b][sl] = cbs[b][sl] + coff
        pltpu.sync_copy(ei_ref.at[0, pl.ds(e0, CHUNK)], rbs[b])
        pltpu.async_copy(s_ref.at[cbs[b]], gbs[b], gsems[b])

    def wait_gather(b):
        pltpu.make_async_copy(s_ref.at[cbs[b]], gbs[b], gsems[b]).wait()

    def start_scatter(b):
        pltpu.async_copy(gbs[b], acc.at[rbs[b]], ssems[b], add=True)

    def wait_scatter(b):
        pltpu.make_async_copy(gbs[b], acc.at[rbs[b]], ssems[b]).wait()

    load_gather(0, 0)
    for b in range(1, NBUF):
        load_gather(b, b)
        wait_gather(b - 1)
        start_scatter(b - 1)

    def outer(t, carry):
        for b in range(NBUF):
            j = t * NBUF + b
            wait_scatter(b)
            load_gather(j, b)
            pb = (b - 1) % NBUF
            wait_gather(pb)
            start_scatter(pb)
        return carry

    lax.fori_loop(1, P_CHUNKS_PER_TEC // NBUF, outer, 0)
    wait_gather(NBUF - 1)
    start_scatter(NBUF - 1)
    for b in range(NBUF):
        wait_scatter(b)
    plsc.subcore_barrier()

    ob = w * OUT_ROWS_PER_TEC
    pltpu.sync_copy(acc.at[pl.ds(ob, OUT_ROWS_PER_TEC)],
                    out_ref.at[pl.ds(coff + ob, OUT_ROWS_PER_TEC)])


_prop_call = functools.partial(
    pl.kernel,
    _prop_body,
    out_type=jax.ShapeDtypeStruct((2 * N_NODES, H), _f32),
    mesh=_MESH,
    scratch_types=(
        [pltpu.VMEM_SHARED((ACC_ROWS, H), _f32), pltpu.VMEM((ZROWS, H), _f32)]
        + [pltpu.VMEM((CHUNK,), _i32) for _ in range(8)]
        + [pltpu.VMEM((CHUNK, H), _f32) for _ in range(4)]
        + [pltpu.SemaphoreType.DMA for _ in range(8)]
    ),
    compiler_params=_SC_PARAMS,
)()


_BLK = 2000
_NB = N_NODES // _BLK


def _dis_body(p_ref, dis_ref):
    deg = p_ref[0, :, 0:1] + p_ref[1, :, 0:1]
    dis_ref[...] = jnp.where(deg > 0.0, lax.rsqrt(jnp.maximum(deg, 1e-30)), 0.0)


def _tc_dis(p):
    return pl.pallas_call(
        _dis_body,
        grid=(_NB,),
        in_specs=[pl.BlockSpec((2, _BLK, H), lambda i: (0, i, 0))],
        out_specs=pl.BlockSpec((_BLK, 1), lambda i: (i, 0)),
        out_shape=jax.ShapeDtypeStruct((N_NODES, 1), _f32),
    )(p)


def _scale0_body(e_ref, dis_ref, s_ref):
    i = pl.program_id(0)
    half = jnp.where(i < _NB, e_ref[:, 0:H], e_ref[:, H:D])
    s_ref[...] = half * dis_ref[...]


def _tc_scale0(e0, dis):
    return pl.pallas_call(
        _scale0_body,
        grid=(2 * _NB,),
        in_specs=[
            pl.BlockSpec((_BLK, D), lambda i: (i % _NB, 0)),
            pl.BlockSpec((_BLK, 1), lambda i: (i % _NB, 0)),
        ],
        out_specs=pl.BlockSpec((_BLK, H), lambda i: (i, 0)),
        out_shape=jax.ShapeDtypeStruct((2 * N_NODES, H), _f32),
    )(e0, dis)


def _rescale_body(a_ref, dis_ref, s_ref):
    d = dis_ref[...]
    s_ref[...] = a_ref[...] * (d * d)


def _tc_rescale(a, dis):
    return pl.pallas_call(
        _rescale_body,
        grid=(2 * _NB,),
        in_specs=[
            pl.BlockSpec((_BLK, H), lambda i: (i, 0)),
            pl.BlockSpec((_BLK, 1), lambda i: (i % _NB, 0)),
        ],
        out_specs=pl.BlockSpec((_BLK, H), lambda i: (i, 0)),
        out_shape=jax.ShapeDtypeStruct((2 * N_NODES, H), _f32),
    )(a, dis)


def _final_body(w_ref, dis_ref, a1l, a1h, a2l, a2h, a3l, a3h, out_ref):
    d = dis_ref[...]
    lo = (a1l[...] + a2l[...] + a3l[...]) * d
    hi = (a1h[...] + a2h[...] + a3h[...]) * d
    out_ref[...] = (w_ref[...] + jnp.concatenate([lo, hi], axis=1)) * 0.25


def _tc_final(weights, dis, a1, a2, a3, node_off_blocks):
    nb = weights.shape[0] // _BLK
    noff = node_off_blocks

    def lo_map(i):
        return (noff + i, 0)

    def hi_map(i):
        return (_NB + noff + i, 0)

    a_specs = []
    for _ in range(3):
        a_specs.append(pl.BlockSpec((_BLK, H), lo_map))
        a_specs.append(pl.BlockSpec((_BLK, H), hi_map))
    return pl.pallas_call(
        _final_body,
        grid=(nb,),
        in_specs=[
            pl.BlockSpec((_BLK, D), lambda i: (i, 0)),
            pl.BlockSpec((_BLK, 1), lo_map),
        ] + a_specs,
        out_specs=pl.BlockSpec((_BLK, D), lambda i: (i, 0)),
        out_shape=jax.ShapeDtypeStruct((weights.shape[0], D), _f32),
    )(weights, dis, a1, a1, a2, a2, a3, a3)


def kernel(user_weight, item_weight, edge_index):
    e0 = jnp.concatenate([user_weight, item_weight], axis=0)
    npad = E_PAD - E
    pad = jnp.stack([
        jnp.full((npad,), N_NODES, _i32),
        jnp.zeros((npad,), _i32),
    ])
    ei = jnp.concatenate([edge_index.astype(_i32), pad], axis=1)

    p = _deg_call(ei)
    dis = _tc_dis(p)
    s = _tc_scale0(e0, dis)
    a1 = _prop_call(s, ei)
    s = _tc_rescale(a1, dis)
    a2 = _prop_call(s, ei)
    s = _tc_rescale(a2, dis)
    a3 = _prop_call(s, ei)

    user_out = _tc_final(user_weight, dis, a1, a2, a3, 0)
    item_out = _tc_final(item_weight, dis, a1, a2, a3, _NB // 2)
    return (user_out, item_out)


# speedup vs baseline: 14.9441x; 1.1857x over previous
import functools

import jax
import jax.numpy as jnp
from jax import lax
from jax.experimental import pallas as pl
from jax.experimental.pallas import tpu as pltpu
from jax.experimental.pallas import tpu_sc as plsc

N_NODES = 100000
D = 32
H = 16
E = 1600000
CHUNK = 128
NBUF = 4
SUP = 8
P_CHUNKS_PER_TEC = 784
E_PAD = P_CHUNKS_PER_TEC * 16 * CHUNK
E_ALLOC = E_PAD + SUP * CHUNK
N_CHUNKS_ALLOC = E_ALLOC // CHUNK
DEG_CHUNKS_PER_TEC = E_PAD // (32 * CHUNK)
ACC_ROWS = N_NODES + 16
ZROWS = 893
ROWS_PER_TEC = ACC_ROWS // 16
OUT_ROWS_PER_TEC = N_NODES // 16

_f32 = jnp.float32
_i32 = jnp.int32

_MESH = plsc.VectorSubcoreMesh(core_axis_name="c", subcore_axis_name="s")
_SC_PARAMS = pltpu.CompilerParams(use_tc_tiling_on_sc=False)


def _zero_acc(acc, zbuf, w):
    def zrow(i, carry):
        zbuf[i, :] = jnp.zeros((16,), _f32)
        return carry

    lax.fori_loop(0, ZROWS, zrow, 0)
    base = w * ROWS_PER_TEC
    for k in range(7):
        pltpu.sync_copy(zbuf, acc.at[pl.ds(base + k * ZROWS, ZROWS)])


def _deg_body(ei_ref, out_ref, acc, zbuf, ones_b, rb0, rb1, rs0, rs1,
              ss0, ss1, ss2, ss3):
    c = lax.axis_index("c")
    w = lax.axis_index("s")
    rbs = (rb0, rb1)
    rsems = (rs0, rs1)
    ssems = (ss0, ss1, ss2, ss3)

    _zero_acc(acc, zbuf, w)

    def orow(i, carry):
        ones_b[i, :] = jnp.ones((16,), _f32)
        return carry

    lax.fori_loop(0, CHUNK, orow, 0)
    plsc.subcore_barrier()

    chunk0 = (c * 16 + w) * DEG_CHUNKS_PER_TEC

    def prefetch(cidx, p):
        pltpu.async_copy(ei_ref.at[0, pl.ds(cidx, SUP)], rbs[p], rsems[p])

    def wait_prefetch(p):
        pltpu.make_async_copy(ei_ref.at[0, pl.ds(0, SUP)], rbs[p],
                              rsems[p]).wait()

    def scatter(p, k, b):
        pltpu.async_copy(ones_b, acc.at[rbs[p].at[k]], ssems[b], add=True)

    def wait_scatter(p, k, b):
        pltpu.make_async_copy(ones_b, acc.at[rbs[p].at[k]], ssems[b]).wait()

    def do_sup(cbase, p, first):
        wait_prefetch(p)
        for k in range(SUP):
            b = k % NBUF
            if not (first and k < NBUF):
                wp, wk = (p, k - NBUF) if k >= NBUF else (1 - p, k + NBUF)
                wait_scatter(wp, wk, b)
            scatter(p, k, b)
            if k == 3:
                prefetch(cbase + SUP, 1 - p)

    prefetch(chunk0, 0)
    do_sup(chunk0, 0, True)

    def outer(i, carry):
        cbase = chunk0 + (1 + 2 * i) * SUP
        do_sup(cbase, 1, False)
        do_sup(cbase + SUP, 0, False)
        return carry

    lax.fori_loop(0, 24, outer, 0)
    for k in range(NBUF, SUP):
        wait_scatter(0, k, k % NBUF)
    wait_prefetch(1)
    plsc.subcore_barrier()

    ob = w * OUT_ROWS_PER_TEC
    pltpu.sync_copy(acc.at[pl.ds(ob, OUT_ROWS_PER_TEC)],
                    out_ref.at[c, pl.ds(ob, OUT_ROWS_PER_TEC)])


_deg_call = functools.partial(
    pl.kernel,
    _deg_body,
    out_type=jax.ShapeDtypeStruct((2, N_NODES, H), _f32),
    mesh=_MESH,
    scratch_types=(
        [pltpu.VMEM_SHARED((ACC_ROWS, H), _f32),
         pltpu.VMEM((ZROWS, H), _f32),
         pltpu.VMEM((CHUNK, H), _f32)]
        + [pltpu.VMEM((SUP, CHUNK), _i32) for _ in range(2)]
        + [pltpu.SemaphoreType.DMA for _ in range(6)]
    ),
    compiler_params=_SC_PARAMS,
)()


def _prop_body(s_ref, ei_ref, out_ref, acc, zbuf,
               cb0, cb1, rb0, rb1,
               ib0, ib1, ib2, ib3,
               gb0, gb1, gb2, gb3,
               cs0, cs1, rs0, rs1,
               gs0, gs1, gs2, gs3, ss0, ss1, ss2, ss3):
    c = lax.axis_index("c")
    w = lax.axis_index("s")
    cbs = (cb0, cb1)
    rbs = (rb0, rb1)
    ibs = (ib0, ib1, ib2, ib3)
    gbs = (gb0, gb1, gb2, gb3)
    csems = (cs0, cs1)
    rsems = (rs0, rs1)
    gsems = (gs0, gs1, gs2, gs3)
    ssems = (ss0, ss1, ss2, ss3)

    _zero_acc(acc, zbuf, w)
    plsc.subcore_barrier()

    coff = c * N_NODES
    chunk0 = w * P_CHUNKS_PER_TEC

    def prefetch(cidx, p):
        pltpu.async_copy(ei_ref.at[1, pl.ds(cidx, SUP)], cbs[p], csems[p])
        pltpu.async_copy(ei_ref.at[0, pl.ds(cidx, SUP)], rbs[p], rsems[p])

    def wait_prefetch(p):
        pltpu.make_async_copy(ei_ref.at[1, pl.ds(0, SUP)], cbs[p],
                              csems[p]).wait()
        pltpu.make_async_copy(ei_ref.at[0, pl.ds(0, SUP)], rbs[p],
                              rsems[p]).wait()

    def gather(b):
        pltpu.async_copy(s_ref.at[ibs[b]], gbs[b], gsems[b])

    def wait_gather(b):
        pltpu.make_async_copy(s_ref.at[ibs[b]], gbs[b], gsems[b]).wait()

    def scatter(p, k, b):
        pltpu.async_copy(gbs[b], acc.at[rbs[p].at[k]], ssems[b], add=True)

    def wait_scatter(p, k, b):
        pltpu.make_async_copy(gbs[b], acc.at[rbs[p].at[k]], ssems[b]).wait()

    def do_sup(cbase, p, first):
        wait_prefetch(p)
        for k in range(SUP):
            b = k % NBUF
            if not (first and k == 0):
                pp, pk = (p, k - 1) if k > 0 else (1 - p, SUP - 1)
                pb = (b - 1) % NBUF
                wait_gather(pb)
                scatter(pp, pk, pb)
            if not (first and k < NBUF):
                wp, wk = (p, k - NBUF) if k >= NBUF else (1 - p, k + NBUF)
                wait_scatter(wp, wk, b)
            for m in range(CHUNK // 16):
                sl = pl.ds(m * 16, 16)
                ibs[b][sl] = cbs[p][k, sl] + coff
            gather(b)
            if k == 3:
                prefetch(cbase + SUP, 1 - p)

    prefetch(chunk0, 0)
    do_sup(chunk0, 0, True)

    def outer(i, carry):
        cbase = chunk0 + (1 + 2 * i) * SUP
        do_sup(cbase, 1, False)
        do_sup(cbase + SUP, 0, False)
        return carry

    lax.fori_loop(0, 48, outer, 0)
    do_sup(chunk0 + 97 * SUP, 1, False)
    wait_gather(3)
    scatter(1, SUP - 1, 3)
    for k in range(NBUF, SUP):
        wait_scatter(1, k, k % NBUF)
    wait_prefetch(0)
    plsc.subcore_barrier()

    ob = w * OUT_ROWS_PER_TEC
    pltpu.sync_copy(acc.at[pl.ds(ob, OUT_ROWS_PER_TEC)],
                    out_ref.at[pl.ds(coff + ob, OUT_ROWS_PER_TEC)])


_prop_call = functools.partial(
    pl.kernel,
    _prop_body,
    out_type=jax.ShapeDtypeStruct((2 * N_NODES, H), _f32),
    mesh=_MESH,
    scratch_types=(
        [pltpu.VMEM_SHARED((ACC_ROWS, H), _f32), pltpu.VMEM((ZROWS, H), _f32)]
        + [pltpu.VMEM((SUP, CHUNK), _i32) for _ in range(4)]
        + [pltpu.VMEM((CHUNK,), _i32) for _ in range(4)]
        + [pltpu.VMEM((CHUNK, H), _f32) for _ in range(4)]
        + [pltpu.SemaphoreType.DMA for _ in range(12)]
    ),
    compiler_params=_SC_PARAMS,
)()


_BLK = 2000
_NB = N_NODES // _BLK


def _dis_body(p_ref, dis_ref):
    deg = p_ref[0, :, 0:1] + p_ref[1, :, 0:1]
    dis_ref[...] = jnp.where(deg > 0.0, lax.rsqrt(jnp.maximum(deg, 1e-30)), 0.0)


def _tc_dis(p):
    return pl.pallas_call(
        _dis_body,
        grid=(_NB,),
        in_specs=[pl.BlockSpec((2, _BLK, H), lambda i: (0, i, 0))],
        out_specs=pl.BlockSpec((_BLK, 1), lambda i: (i, 0)),
        out_shape=jax.ShapeDtypeStruct((N_NODES, 1), _f32),
    )(p)


def _scale0_body(e_ref, dis_ref, s_ref):
    i = pl.program_id(0)
    half = jnp.where(i < _NB, e_ref[:, 0:H], e_ref[:, H:D])
    s_ref[...] = half * dis_ref[...]


def _tc_scale0(e0, dis):
    return pl.pallas_call(
        _scale0_body,
        grid=(2 * _NB,),
        in_specs=[
            pl.BlockSpec((_BLK, D), lambda i: (i % _NB, 0)),
            pl.BlockSpec((_BLK, 1), lambda i: (i % _NB, 0)),
        ],
        out_specs=pl.BlockSpec((_BLK, H), lambda i: (i, 0)),
        out_shape=jax.ShapeDtypeStruct((2 * N_NODES, H), _f32),
    )(e0, dis)


def _rescale_body(a_ref, dis_ref, s_ref):
    d = dis_ref[...]
    s_ref[...] = a_ref[...] * (d * d)


def _tc_rescale(a, dis):
    return pl.pallas_call(
        _rescale_body,
        grid=(2 * _NB,),
        in_specs=[
            pl.BlockSpec((_BLK, H), lambda i: (i, 0)),
            pl.BlockSpec((_BLK, 1), lambda i: (i % _NB, 0)),
        ],
        out_specs=pl.BlockSpec((_BLK, H), lambda i: (i, 0)),
        out_shape=jax.ShapeDtypeStruct((2 * N_NODES, H), _f32),
    )(a, dis)


def _final_body(w_ref, dis_ref, a1l, a1h, a2l, a2h, a3l, a3h, out_ref):
    d = dis_ref[...]
    lo = (a1l[...] + a2l[...] + a3l[...]) * d
    hi = (a1h[...] + a2h[...] + a3h[...]) * d
    out_ref[...] = (w_ref[...] + jnp.concatenate([lo, hi], axis=1)) * 0.25


def _tc_final(weights, dis, a1, a2, a3, node_off_blocks):
    nb = weights.shape[0] // _BLK
    noff = node_off_blocks

    def lo_map(i):
        return (noff + i, 0)

    def hi_map(i):
        return (_NB + noff + i, 0)

    a_specs = []
    for _ in range(3):
        a_specs.append(pl.BlockSpec((_BLK, H), lo_map))
        a_specs.append(pl.BlockSpec((_BLK, H), hi_map))
    return pl.pallas_call(
        _final_body,
        grid=(nb,),
        in_specs=[
            pl.BlockSpec((_BLK, D), lambda i: (i, 0)),
            pl.BlockSpec((_BLK, 1), lo_map),
        ] + a_specs,
        out_specs=pl.BlockSpec((_BLK, D), lambda i: (i, 0)),
        out_shape=jax.ShapeDtypeStruct((weights.shape[0], D), _f32),
    )(weights, dis, a1, a1, a2, a2, a3, a3)


def kernel(user_weight, item_weight, edge_index):
    e0 = jnp.concatenate([user_weight, item_weight], axis=0)
    npad = E_ALLOC - E
    pad = jnp.stack([
        jnp.full((npad,), N_NODES, _i32),
        jnp.zeros((npad,), _i32),
    ])
    ei = jnp.concatenate([edge_index.astype(_i32), pad], axis=1)
    ei = ei.reshape(2, N_CHUNKS_ALLOC, CHUNK)

    p = _deg_call(ei)
    dis = _tc_dis(p)
    s = _tc_scale0(e0, dis)
    a1 = _prop_call(s, ei)
    s = _tc_rescale(a1, dis)
    a2 = _prop_call(s, ei)
    s = _tc_rescale(a2, dis)
    a3 = _prop_call(s, ei)

    user_out = _tc_final(user_weight, dis, a1, a2, a3, 0)
    item_out = _tc_final(item_weight, dis, a1, a2, a3, _NB // 2)
    return (user_out, item_out)
